# Initial kernel scaffold; baseline (speedup 1.0000x reference)
import jax
import jax.numpy as jnp
from jax.experimental import pallas as pl


def kernel(inputs):
    raise NotImplementedError("write your pallas kernel here")


# speedup vs baseline: 159.7935x; 159.7935x over previous
import functools

import jax
import jax.numpy as jnp
from jax.experimental import pallas as pl
from jax.experimental.pallas import tpu as pltpu

K_ROWS = 256
N_COLS = 8192
NEG = float("-inf")


def _tc_body(x_ref, o_ref):
    x = x_ref[...]
    m1 = jnp.max(x, axis=1, keepdims=True)
    c1 = jnp.sum((x == m1).astype(jnp.float32), axis=1, keepdims=True)
    x2 = jnp.where(x < m1, x, NEG)
    m2 = jnp.max(x2, axis=1, keepdims=True)
    c2 = jnp.sum((x == m2).astype(jnp.float32), axis=1, keepdims=True)
    x3 = jnp.where(x2 < m2, x2, NEG)
    m3 = jnp.max(x3, axis=1, keepdims=True)
    kth = jnp.where(c1 >= 3.0, m1, jnp.where(c1 + c2 >= 3.0, m2, m3))
    v = jnp.where(x >= kth, x, 0.0)
    s = jnp.sum(v, axis=1, keepdims=True)
    o_ref[...] = v * (1.0 / s)


@jax.jit
def kernel(inputs):
    b, h, n = inputs.shape
    x2d = inputs.reshape(b * h, n)
    rows = b * h
    out = pl.pallas_call(
        _tc_body,
        grid=(rows // K_ROWS,),
        in_specs=[pl.BlockSpec((K_ROWS, n), lambda i: (i, 0))],
        out_specs=pl.BlockSpec((K_ROWS, n), lambda i: (i, 0)),
        out_shape=jax.ShapeDtypeStruct((rows, n), jnp.float32),
    )(x2d)
    return out.reshape(b, h, n)
